# baseline (speedup 1.0000x reference)
import functools

import jax
import jax.numpy as jnp
from jax import lax
from jax.experimental import pallas as pl
from jax.experimental.pallas import tpu as pltpu
from jax.experimental.pallas import tpu_sc as plsc

B, Ts, Tw = 32, 128, 20
CHAR_V, WORD_V, F, C = 100, 100000, 32, 15
CE_DIM, WE_DIM, GE = 32, 300, 16
CH_H, WH = 512, 512
BT = B * Ts
NCHARS = BT * Tw
WE_SC = 256
WE_TAIL = WE_DIM - WE_SC
C_PAD = 16

_NC, _NS = 2, 16
NW = _NC * _NS
BT_PER_W = BT // NW
CH_PER_W = NCHARS // NW
CH_CHUNK = 128
N_CHUNKS = CH_PER_W // CH_CHUNK

@functools.cache
def _sc_kernels():
    mesh = plsc.VectorSubcoreMesh(core_axis_name="c", subcore_axis_name="s")
    cparams = pltpu.CompilerParams(use_tc_tiling_on_sc=False,
                                   needs_layout_passes=False)
    cparams_tc = pltpu.CompilerParams(needs_layout_passes=False)

    @functools.partial(
        pl.kernel,
        mesh=mesh,
        out_type=jax.ShapeDtypeStruct((BT, WE_SC), jnp.float32),
        scratch_types=[
            pltpu.VMEM((BT_PER_W,), jnp.int32),
            pltpu.VMEM((BT_PER_W, WE_SC), jnp.float32),
            pltpu.SemaphoreType.DMA,
        ],
        compiler_params=cparams_tc,
    )
    def sc_gather_we(table_hbm, idx_hbm, out_hbm, idx_v, rows_v, sem):
        wid = lax.axis_index("s") * _NC + lax.axis_index("c")
        base = wid * BT_PER_W
        pltpu.sync_copy(idx_hbm.at[pl.ds(base, BT_PER_W)], idx_v)
        cps = [
            pltpu.async_copy(table_hbm.at[idx_v, pl.ds(0, 128)],
                             rows_v.at[:, pl.ds(0, 128)], sem),
            pltpu.async_copy(table_hbm.at[idx_v, pl.ds(128, 128)],
                             rows_v.at[:, pl.ds(128, 128)], sem),
        ]
        for cp in cps:
            cp.wait()
        pltpu.sync_copy(rows_v, out_hbm.at[pl.ds(base, BT_PER_W)])

    @functools.partial(
        pl.kernel,
        mesh=mesh,
        out_type=jax.ShapeDtypeStruct((NCHARS, C_PAD), jnp.float32),
        scratch_types=[
            pltpu.VMEM((CH_PER_W,), jnp.int32),
            pltpu.VMEM((BT_PER_W,), jnp.int32),
            pltpu.VMEM((CH_PER_W,), jnp.int32),
            pltpu.VMEM((CH_PER_W, C_PAD), jnp.float32),
            pltpu.SemaphoreType.DMA,
        ],
        compiler_params=cparams,
    )
    def sc_combine(t_hbm, cids_hbm, wl_hbm, out_hbm,
                   cid_v, wl_v, idx_v, rows_v, sem):
        wid = lax.axis_index("s") * _NC + lax.axis_index("c")
        base_bt = wid * BT_PER_W
        base_f = wid * CH_PER_W
        pltpu.sync_copy(cids_hbm.at[pl.ds(base_f, CH_PER_W)], cid_v)
        pltpu.sync_copy(wl_hbm.at[pl.ds(base_bt, BT_PER_W)], wl_v)

        def body(i, carry):
            cid = cid_v[pl.ds(i * 16, 16)]
            idx_v[pl.ds(i * 16, 16)] = BT + cid
            return carry

        lax.fori_loop(0, CH_PER_W // 16, body, 0)

        def body2(i, carry):
            bt_l = lax.iota(jnp.int32, 16) + i * 16
            lw = wl_v[pl.ds(i * 16, 16)]
            hitpos = bt_l * Tw + jnp.maximum(lw - 1, 0)
            plsc.store_scatter(idx_v, [hitpos], base_bt + bt_l, mask=lw > 0)
            return carry

        lax.fori_loop(0, BT_PER_W // 16, body2, 0)

        copies = []
        for j in range(N_CHUNKS):
            copies.append(pltpu.async_copy(
                t_hbm.at[idx_v.at[pl.ds(j * CH_CHUNK, CH_CHUNK)]],
                rows_v.at[pl.ds(j * CH_CHUNK, CH_CHUNK)],
                sem,
            ))
        for cp in copies:
            cp.wait()
        pltpu.sync_copy(rows_v, out_hbm.at[pl.ds(base_f, CH_PER_W)])

    return sc_gather_we, sc_combine


TB = 512
GRID = BT // TB


def _tc_body(we_ref, wet_ref, posf_ref, g_ref, n_ref, p_ref,
             ct_ref, w1_ref, b1_ref, w2_ref, b2_ref,
             wcwa_ref, wcwb_ref, wcf_ref, wcg_ref, wcn_ref, wcp_ref, bc_ref,
             wo_ref, bo_ref, gt_ref, nt_ref, pt_ref,
             upd_ref, lch_ref):
    f32 = jnp.float32

    def pad_c(x):
        return jnp.concatenate(
            [x, jnp.zeros((x.shape[0], C_PAD - C), f32)], axis=1)

    ch_h = jnp.tanh(jnp.dot(ct_ref[...], w1_ref[...],
                            preferred_element_type=f32) + b1_ref[...])
    lch_ref[...] = pad_c(jnp.dot(ch_h, w2_ref[...],
                                 preferred_element_type=f32) + b2_ref[...])
    mg = jnp.dot(gt_ref[...], wcg_ref[...], preferred_element_type=f32)
    mn = jnp.dot(nt_ref[...], wcn_ref[...], preferred_element_type=f32)
    mp = jnp.dot(pt_ref[...], wcp_ref[...], preferred_element_type=f32)
    ohg = (g_ref[...] == lax.broadcasted_iota(jnp.int32, (1, 3), 1)).astype(f32)
    ohn = (n_ref[...] == lax.broadcasted_iota(jnp.int32, (1, 3), 1)).astype(f32)
    ohp = (p_ref[...] == lax.broadcasted_iota(jnp.int32, (1, 4), 1)).astype(f32)
    hpre = (jnp.dot(we_ref[...], wcwa_ref[...], preferred_element_type=f32)
            + jnp.dot(wet_ref[...], wcwb_ref[...], preferred_element_type=f32)
            + jnp.dot(posf_ref[...], wcf_ref[...], preferred_element_type=f32)
            + jnp.dot(ohg, mg, preferred_element_type=f32)
            + jnp.dot(ohn, mn, preferred_element_type=f32)
            + jnp.dot(ohp, mp, preferred_element_type=f32)
            + bc_ref[...])
    hc = jnp.tanh(hpre)
    upd_ref[...] = pad_c(jnp.dot(hc, wo_ref[...],
                                 preferred_element_type=f32) + bo_ref[...])


def _full(shape):
    return pl.BlockSpec(shape, lambda i: (0,) * len(shape))


def _tc_ce(we, wet, posf, gid, nid, pid, ct, w1, b1, w2, b2,
           wcwa, wcwb, wcf, wcg, wcn, wcp, bc, wo, bo, gt, nt, pt):
    return pl.pallas_call(
        _tc_body,
        grid=(GRID,),
        in_specs=[
            pl.BlockSpec((TB, WE_SC), lambda i: (i, 0)),
            pl.BlockSpec((TB, WE_TAIL), lambda i: (i, 0)),
            pl.BlockSpec((TB, F), lambda i: (i, 0)),
            pl.BlockSpec((TB, 1), lambda i: (i, 0)),
            pl.BlockSpec((TB, 1), lambda i: (i, 0)),
            pl.BlockSpec((TB, 1), lambda i: (i, 0)),
            _full((CHAR_V, CE_DIM)), _full((CE_DIM, CH_H)), _full((1, CH_H)),
            _full((CH_H, C)), _full((1, C)),
            _full((WE_SC, WH)), _full((WE_TAIL, WH)), _full((F, WH)),
            _full((GE, WH)), _full((GE, WH)), _full((GE, WH)), _full((1, WH)),
            _full((WH, C)), _full((1, C)),
            _full((3, GE)), _full((3, GE)), _full((4, GE)),
        ],
        out_specs=[
            pl.BlockSpec((TB, C_PAD), lambda i: (i, 0)),
            pl.BlockSpec((CHAR_V, C_PAD), lambda i: (0, 0)),
        ],
        out_shape=[
            jax.ShapeDtypeStruct((BT, C_PAD), jnp.float32),
            jax.ShapeDtypeStruct((CHAR_V, C_PAD), jnp.float32),
        ],
    )(we, wet, posf, gid, nid, pid, ct, w1, b1, w2, b2,
      wcwa, wcwb, wcf, wcg, wcn, wcp, bc, wo, bo, gt, nt, pt)


def kernel(word_ids, char_ids, pos_features, gender_ids, number_ids,
           person_ids, word_lengths, char_table, W1, b1, W2, b2,
           word_table, gender_table, number_table, person_table,
           Wc, bc, Wo, bo):
    wid_flat = word_ids.reshape(BT).astype(jnp.int32)
    cids_flat = char_ids.reshape(NCHARS).astype(jnp.int32)
    wl_flat = word_lengths.reshape(BT).astype(jnp.int32)
    posf = pos_features.reshape(BT, F)
    gid = gender_ids.reshape(BT, 1).astype(jnp.int32)
    nid = number_ids.reshape(BT, 1).astype(jnp.int32)
    pid = person_ids.reshape(BT, 1).astype(jnp.int32)

    wcwa = Wc[:WE_SC]
    wcwb = Wc[WE_SC:WE_DIM]
    wcf = Wc[WE_DIM:WE_DIM + F]
    wcg = Wc[WE_DIM + F:WE_DIM + F + GE]
    wcn = Wc[WE_DIM + F + GE:WE_DIM + F + 2 * GE]
    wcp = Wc[WE_DIM + F + 2 * GE:]

    sc_gather_we, sc_combine = _sc_kernels()
    we = sc_gather_we(word_table, wid_flat)
    tail_tbl = jax.lax.optimization_barrier(word_table[:, WE_SC:WE_DIM])
    wet = jnp.take(tail_tbl, wid_flat, axis=0)

    upd, lch = _tc_ce(
        we, wet, posf, gid, nid, pid, char_table, W1, b1.reshape(1, CH_H), W2,
        b2.reshape(1, C), wcwa, wcwb, wcf, wcg, wcn, wcp, bc.reshape(1, WH),
        Wo, bo.reshape(1, C), gender_table, number_table, person_table)

    t = jnp.concatenate([upd, lch], axis=0)
    out = sc_combine(t, cids_flat, wl_flat)
    return out[:, :C].reshape(B, Ts, Tw, C)


# speedup vs baseline: 1.0875x; 1.0875x over previous
import functools

import jax
import jax.numpy as jnp
from jax import lax
from jax.experimental import pallas as pl
from jax.experimental.pallas import tpu as pltpu
from jax.experimental.pallas import tpu_sc as plsc

B, Ts, Tw = 32, 128, 20
CHAR_V, WORD_V, F, C = 100, 100000, 32, 15
CE_DIM, WE_DIM, GE = 32, 300, 16
CH_H, WH = 512, 512
BT = B * Ts
NCHARS = BT * Tw
WE_SC = 256
WE_TAIL = WE_DIM - WE_SC
C_PAD = 16

_NC, _NS = 2, 16
NW = _NC * _NS
BT_PER_W = BT // NW
CH_PER_W = NCHARS // NW
CH_CHUNK = 128
N_CHUNKS = CH_PER_W // CH_CHUNK

@functools.cache
def _sc_kernels():
    mesh = plsc.VectorSubcoreMesh(core_axis_name="c", subcore_axis_name="s")
    cparams = pltpu.CompilerParams(use_tc_tiling_on_sc=False,
                                   needs_layout_passes=False)
    cparams_tc = pltpu.CompilerParams(needs_layout_passes=False)

    @functools.partial(
        pl.kernel,
        mesh=mesh,
        out_type=jax.ShapeDtypeStruct((BT, WE_SC), jnp.float32),
        scratch_types=[
            pltpu.VMEM((BT_PER_W,), jnp.int32),
            pltpu.VMEM((BT_PER_W, WE_SC), jnp.float32),
            pltpu.SemaphoreType.DMA,
        ],
        compiler_params=cparams_tc,
    )
    def sc_gather_we(table_hbm, idx_hbm, out_hbm, idx_v, rows_v, sem):
        wid = lax.axis_index("s") * _NC + lax.axis_index("c")
        base = wid * BT_PER_W
        pltpu.sync_copy(idx_hbm.at[pl.ds(base, BT_PER_W)], idx_v)
        cps = [
            pltpu.async_copy(table_hbm.at[idx_v, pl.ds(0, 128)],
                             rows_v.at[:, pl.ds(0, 128)], sem),
            pltpu.async_copy(table_hbm.at[idx_v, pl.ds(128, 128)],
                             rows_v.at[:, pl.ds(128, 128)], sem),
        ]
        for cp in cps:
            cp.wait()
        pltpu.sync_copy(rows_v, out_hbm.at[pl.ds(base, BT_PER_W)])

    @functools.partial(
        pl.kernel,
        mesh=mesh,
        out_type=jax.ShapeDtypeStruct((NCHARS, C_PAD), jnp.float32),
        scratch_types=[
            pltpu.VMEM((CH_PER_W,), jnp.int32),
            pltpu.VMEM((BT_PER_W,), jnp.int32),
            pltpu.VMEM((CH_PER_W,), jnp.int32),
            pltpu.VMEM((CH_PER_W, C_PAD), jnp.float32),
            pltpu.SemaphoreType.DMA,
        ],
        compiler_params=cparams,
    )
    def sc_combine(t_hbm, cids_hbm, wl_hbm, out_hbm,
                   cid_v, wl_v, idx_v, rows_v, sem):
        wid = lax.axis_index("s") * _NC + lax.axis_index("c")
        base_bt = wid * BT_PER_W
        base_f = wid * CH_PER_W
        pltpu.sync_copy(cids_hbm.at[pl.ds(base_f, CH_PER_W)], cid_v)
        pltpu.sync_copy(wl_hbm.at[pl.ds(base_bt, BT_PER_W)], wl_v)

        def body(i, carry):
            cid = cid_v[pl.ds(i * 16, 16)]
            idx_v[pl.ds(i * 16, 16)] = BT + cid
            return carry

        lax.fori_loop(0, CH_PER_W // 16, body, 0)

        def body2(i, carry):
            bt_l = lax.iota(jnp.int32, 16) + i * 16
            lw = wl_v[pl.ds(i * 16, 16)]
            hitpos = bt_l * Tw + jnp.maximum(lw - 1, 0)
            plsc.store_scatter(idx_v, [hitpos], base_bt + bt_l, mask=lw > 0)
            return carry

        lax.fori_loop(0, BT_PER_W // 16, body2, 0)

        copies = []
        for j in range(N_CHUNKS):
            copies.append(pltpu.async_copy(
                t_hbm.at[idx_v.at[pl.ds(j * CH_CHUNK, CH_CHUNK)]],
                rows_v.at[pl.ds(j * CH_CHUNK, CH_CHUNK)],
                sem,
            ))
        for cp in copies:
            cp.wait()
        pltpu.sync_copy(rows_v, out_hbm.at[pl.ds(base_f, CH_PER_W)])

    return sc_gather_we, sc_combine


TB = 512
GRID = BT // TB


def _tc_body(we_ref, wet_ref, posf_ref, g_ref, n_ref, p_ref,
             ct_ref, w1_ref, b1_ref, w2_ref, b2_ref,
             wcwa_ref, wcwb_ref, wcf_ref, wcg_ref, wcn_ref, wcp_ref, bc_ref,
             wo_ref, bo_ref, gt_ref, nt_ref, pt_ref,
             upd_ref, lch_ref):
    f32 = jnp.float32

    def pad_c(x):
        return jnp.concatenate(
            [x, jnp.zeros((x.shape[0], C_PAD - C), f32)], axis=1)

    ch_h = jnp.tanh(jnp.dot(ct_ref[...], w1_ref[...],
                            preferred_element_type=f32) + b1_ref[...])
    lch_ref[...] = pad_c(jnp.dot(ch_h, w2_ref[...],
                                 preferred_element_type=f32) + b2_ref[...])
    mg = jnp.dot(gt_ref[...], wcg_ref[...], preferred_element_type=f32)
    mn = jnp.dot(nt_ref[...], wcn_ref[...], preferred_element_type=f32)
    mp = jnp.dot(pt_ref[...], wcp_ref[...], preferred_element_type=f32)
    ohg = (g_ref[...] == lax.broadcasted_iota(jnp.int32, (1, 3), 1)).astype(f32)
    ohn = (n_ref[...] == lax.broadcasted_iota(jnp.int32, (1, 3), 1)).astype(f32)
    ohp = (p_ref[...] == lax.broadcasted_iota(jnp.int32, (1, 4), 1)).astype(f32)
    hpre = (jnp.dot(we_ref[...], wcwa_ref[...], preferred_element_type=f32)
            + jnp.dot(wet_ref[...], wcwb_ref[...], preferred_element_type=f32)
            + jnp.dot(posf_ref[...], wcf_ref[...], preferred_element_type=f32)
            + jnp.dot(ohg, mg, preferred_element_type=f32)
            + jnp.dot(ohn, mn, preferred_element_type=f32)
            + jnp.dot(ohp, mp, preferred_element_type=f32)
            + bc_ref[...])
    hc = jnp.tanh(hpre)
    upd_ref[...] = pad_c(jnp.dot(hc, wo_ref[...],
                                 preferred_element_type=f32) + bo_ref[...])


def _full(shape):
    return pl.BlockSpec(shape, lambda i: (0,) * len(shape))


def _tc_ce(we, wet, posf, gid, nid, pid, ct, w1, b1, w2, b2,
           wcwa, wcwb, wcf, wcg, wcn, wcp, bc, wo, bo, gt, nt, pt):
    return pl.pallas_call(
        _tc_body,
        grid=(GRID,),
        in_specs=[
            pl.BlockSpec((TB, WE_SC), lambda i: (i, 0)),
            pl.BlockSpec((TB, WE_TAIL), lambda i: (i, 0)),
            pl.BlockSpec((TB, F), lambda i: (i, 0)),
            pl.BlockSpec((TB, 1), lambda i: (i, 0)),
            pl.BlockSpec((TB, 1), lambda i: (i, 0)),
            pl.BlockSpec((TB, 1), lambda i: (i, 0)),
            _full((CHAR_V, CE_DIM)), _full((CE_DIM, CH_H)), _full((1, CH_H)),
            _full((CH_H, C)), _full((1, C)),
            _full((WE_SC, WH)), _full((WE_TAIL, WH)), _full((F, WH)),
            _full((GE, WH)), _full((GE, WH)), _full((GE, WH)), _full((1, WH)),
            _full((WH, C)), _full((1, C)),
            _full((3, GE)), _full((3, GE)), _full((4, GE)),
        ],
        out_specs=[
            pl.BlockSpec((TB, C_PAD), lambda i: (i, 0)),
            pl.BlockSpec((CHAR_V, C_PAD), lambda i: (0, 0)),
        ],
        out_shape=[
            jax.ShapeDtypeStruct((BT, C_PAD), jnp.float32),
            jax.ShapeDtypeStruct((CHAR_V, C_PAD), jnp.float32),
        ],
    )(we, wet, posf, gid, nid, pid, ct, w1, b1, w2, b2,
      wcwa, wcwb, wcf, wcg, wcn, wcp, bc, wo, bo, gt, nt, pt)


def kernel(word_ids, char_ids, pos_features, gender_ids, number_ids,
           person_ids, word_lengths, char_table, W1, b1, W2, b2,
           word_table, gender_table, number_table, person_table,
           Wc, bc, Wo, bo):
    wid_flat = word_ids.reshape(BT).astype(jnp.int32)
    cids_flat = char_ids.reshape(NCHARS).astype(jnp.int32)
    wl_flat = word_lengths.reshape(BT).astype(jnp.int32)
    posf = pos_features.reshape(BT, F)
    gid = gender_ids.reshape(BT, 1).astype(jnp.int32)
    nid = number_ids.reshape(BT, 1).astype(jnp.int32)
    pid = person_ids.reshape(BT, 1).astype(jnp.int32)

    wcwa = Wc[:WE_SC]
    wcwb = Wc[WE_SC:WE_DIM]
    wcf = Wc[WE_DIM:WE_DIM + F]
    wcg = Wc[WE_DIM + F:WE_DIM + F + GE]
    wcn = Wc[WE_DIM + F + GE:WE_DIM + F + 2 * GE]
    wcp = Wc[WE_DIM + F + 2 * GE:]

    sc_gather_we, sc_combine = _sc_kernels()
    we = sc_gather_we(word_table, wid_flat)
    live = (wid_flat[:1] >= 0).astype(jnp.float32)
    we_full = jnp.take(word_table, wid_flat, axis=0) * live[:, None]
    we_full = jax.lax.optimization_barrier(we_full)
    wet = we_full[:, WE_SC:WE_DIM]

    upd, lch = _tc_ce(
        we, wet, posf, gid, nid, pid, char_table, W1, b1.reshape(1, CH_H), W2,
        b2.reshape(1, C), wcwa, wcwb, wcf, wcg, wcn, wcp, bc.reshape(1, WH),
        Wo, bo.reshape(1, C), gender_table, number_table, person_table)

    t = jnp.concatenate([upd, lch], axis=0)
    out = sc_combine(t, cids_flat, wl_flat)
    return out[:, :C].reshape(B, Ts, Tw, C)
